# Initial kernel scaffold; baseline (speedup 1.0000x reference)
import jax
import jax.numpy as jnp
from jax.experimental import pallas as pl


def kernel(x, B1, B2, W0, W1, W2):
    raise NotImplementedError("write your pallas kernel here")


# speedup vs baseline: 1.1850x; 1.1850x over previous
import functools

import jax
import jax.numpy as jnp
from jax.experimental import pallas as pl

_BLK_E = 512


def _phase_a_kernel(x_ref, b1_ref, b2_ref, w0_ref, w2_ref, t_ref, n_ref):
    i = pl.program_id(0)
    xb = x_ref[...]
    tb = jax.lax.dot_general(
        b2_ref[...], xb, (((0,), (0,)), ((), ())),
        preferred_element_type=jnp.float32)
    nb = jnp.dot(b1_ref[...], xb, preferred_element_type=jnp.float32)

    @pl.when(i == 0)
    def _init():
        t_ref[...] = tb
        n_ref[...] = nb

    @pl.when(i > 0)
    def _acc():
        t_ref[...] += tb
        n_ref[...] += nb

    @pl.when(i == pl.num_programs(0) - 1)
    def _fold():
        t_ref[...] = jnp.dot(t_ref[...], w2_ref[...],
                             preferred_element_type=jnp.float32)
        n_ref[...] = jnp.dot(n_ref[...], w0_ref[...],
                             preferred_element_type=jnp.float32)


def _phase_b_kernel(x_ref, b1_ref, b2_ref, t_ref, n_ref, w1_ref, o_ref):
    d2 = jnp.dot(b2_ref[...], t_ref[...], preferred_element_type=jnp.float32)
    d0 = jax.lax.dot_general(
        b1_ref[...], n_ref[...], (((0,), (0,)), ((), ())),
        preferred_element_type=jnp.float32)
    d1 = jnp.dot(x_ref[...], w1_ref[...], preferred_element_type=jnp.float32)
    o_ref[...] = jnp.maximum(d2 + d1 + d0, 0.0)


@functools.partial(jax.jit, static_argnames=("interpret",))
def kernel(x, B1, B2, W0, W1, W2, interpret=False):
    n_edges, in_f = x.shape
    n_nodes = B1.shape[0]
    n_tri = B2.shape[1]
    out_f = W0.shape[1]
    grid = (n_edges // _BLK_E,)

    t, n = pl.pallas_call(
        _phase_a_kernel,
        grid=grid,
        in_specs=[
            pl.BlockSpec((_BLK_E, in_f), lambda i: (i, 0)),
            pl.BlockSpec((n_nodes, _BLK_E), lambda i: (0, i)),
            pl.BlockSpec((_BLK_E, n_tri), lambda i: (i, 0)),
            pl.BlockSpec((in_f, out_f), lambda i: (0, 0)),
            pl.BlockSpec((in_f, out_f), lambda i: (0, 0)),
        ],
        out_specs=[
            pl.BlockSpec((n_tri, out_f), lambda i: (0, 0)),
            pl.BlockSpec((n_nodes, out_f), lambda i: (0, 0)),
        ],
        out_shape=[
            jax.ShapeDtypeStruct((n_tri, out_f), jnp.float32),
            jax.ShapeDtypeStruct((n_nodes, out_f), jnp.float32),
        ],
        interpret=interpret,
    )(x, B1, B2, W0, W2)

    out = pl.pallas_call(
        _phase_b_kernel,
        grid=grid,
        in_specs=[
            pl.BlockSpec((_BLK_E, in_f), lambda i: (i, 0)),
            pl.BlockSpec((n_nodes, _BLK_E), lambda i: (0, i)),
            pl.BlockSpec((_BLK_E, n_tri), lambda i: (i, 0)),
            pl.BlockSpec((n_tri, out_f), lambda i: (0, 0)),
            pl.BlockSpec((n_nodes, out_f), lambda i: (0, 0)),
            pl.BlockSpec((in_f, out_f), lambda i: (0, 0)),
        ],
        out_specs=pl.BlockSpec((_BLK_E, out_f), lambda i: (i, 0)),
        out_shape=jax.ShapeDtypeStruct((n_edges, out_f), jnp.float32),
        interpret=interpret,
    )(x, B1, B2, t, n, W1)
    return out
